# baseline (speedup 1.0000x reference)
import jax
import jax.numpy as jnp
from jax.experimental import pallas as pl
from jax.experimental.pallas import tpu as pltpu

N = 10000
D = 128
BI = 400


def _support_step(x_ref, w_ref, sup_ref):
    sup_ref[...] = jnp.dot(x_ref[...], w_ref[...].T, preferred_element_type=jnp.float32)


def _spmm_step(sup_ref, b_ref, adj_ref, out_ref):
    out_ref[...] = (
        jnp.dot(adj_ref[...], sup_ref[...], preferred_element_type=jnp.float32)
        + b_ref[...]
    )


@jax.jit
def kernel(X_input, adj, W, bias):
    bias2d = bias.reshape(1, D)
    support = pl.pallas_call(
        _support_step,
        out_shape=jax.ShapeDtypeStruct((N, D), jnp.float32),
    )(X_input, W)
    return pl.pallas_call(
        _spmm_step,
        grid=(N // BI,),
        in_specs=[
            pl.BlockSpec((N, D), lambda i: (0, 0)),
            pl.BlockSpec((1, D), lambda i: (0, 0)),
            pl.BlockSpec((BI, N), lambda i: (i, 0)),
        ],
        out_specs=pl.BlockSpec((BI, D), lambda i: (i, 0)),
        out_shape=jax.ShapeDtypeStruct((N, D), jnp.float32),
        compiler_params=pltpu.CompilerParams(
            dimension_semantics=("parallel",),
        ),
    )(support, bias2d, adj)
